# baseline (speedup 1.0000x reference)
import jax
import jax.numpy as jnp
from jax.experimental import pallas as pl
from jax.experimental.pallas import tpu as pltpu

BATCH = 8
N_TILES = 4
N_TOKENS = 1025
EMBED_DIM = 1280
MAX_NUM_TILES = 4

NCH = 2
CH = 513


def _body(ar_ref, gate_ref, x_ref, emb_ref, out_ref):
    b = pl.program_id(0)
    h = ar_ref[b, 0, 0]
    w = ar_ref[b, 1, 0]
    n = h * w
    w_safe = jnp.maximum(w, 1)
    gate_t = jnp.tanh(gate_ref[0])
    rows = []
    for t in range(N_TILES):
        valid = t < n
        i = jnp.where(valid, t // w_safe, 0)
        j = jnp.where(valid, t % w_safe, 0)
        row = emb_ref[i, j]
        coef = jnp.where(valid, gate_t, 0.0)
        rows.append(coef * row)
    add = jnp.concatenate(rows, axis=0)

    out_ref[...] = x_ref[...] + add[None, None, :, :]


def kernel(x, aspect_ratio, embedding, gate):
    ar = jax.lax.bitcast_convert_type(aspect_ratio, jnp.int32)
    xt = jnp.transpose(x, (0, 2, 1, 3))

    grid_spec = pltpu.PrefetchScalarGridSpec(
        num_scalar_prefetch=2,
        grid=(BATCH, NCH, 1, 1),
        in_specs=[
            pl.BlockSpec((1, CH, N_TILES, EMBED_DIM),
                         lambda b, c, z0, z1, ar, g: (b, c, z0, z1)),
            pl.BlockSpec((MAX_NUM_TILES, MAX_NUM_TILES, 1, EMBED_DIM),
                         lambda b, c, z0, z1, ar, g: (z0, z1, z0, z1)),
        ],
        out_specs=pl.BlockSpec((1, CH, N_TILES, EMBED_DIM),
                               lambda b, c, z0, z1, ar, g: (b, c, z0, z1)),
    )

    out = pl.pallas_call(
        _body,
        grid_spec=grid_spec,
        out_shape=jax.ShapeDtypeStruct(xt.shape, xt.dtype),
    )(ar, gate.astype(jnp.float32), xt, embedding)
    return jnp.transpose(out, (0, 2, 1, 3))


# speedup vs baseline: 1.0022x; 1.0022x over previous
import jax
import jax.numpy as jnp
from jax.experimental import pallas as pl
from jax.experimental.pallas import tpu as pltpu

BATCH = 8
N_TILES = 4
N_TOKENS = 1025
EMBED_DIM = 1280
MAX_NUM_TILES = 4

NCH = 2
CH = 513


def _body(ar_ref, gate_ref, x_ref, emb_ref, out_ref):
    b = pl.program_id(0)
    h = ar_ref[b, 0]
    w = ar_ref[b, 1]
    n = h * w
    w_safe = jnp.maximum(w, 1)
    gate_t = jnp.tanh(gate_ref[0])
    rows = []
    for t in range(N_TILES):
        valid = t < n
        i = jnp.where(valid, t // w_safe, 0)
        j = jnp.where(valid, t % w_safe, 0)
        row = emb_ref[i, j]
        coef = jnp.where(valid, gate_t, 0.0)
        rows.append(coef * row)
    add = jnp.concatenate(rows, axis=0)

    out_ref[...] = x_ref[...] + add[None, None, :, :]


def kernel(x, aspect_ratio, embedding, gate):
    ar = aspect_ratio.astype(jnp.int32)
    xt = jnp.transpose(x, (0, 2, 1, 3))

    grid_spec = pltpu.PrefetchScalarGridSpec(
        num_scalar_prefetch=2,
        grid=(BATCH, NCH, 1, 1),
        in_specs=[
            pl.BlockSpec((1, CH, N_TILES, EMBED_DIM),
                         lambda b, c, z0, z1, ar, g: (b, c, z0, z1)),
            pl.BlockSpec((MAX_NUM_TILES, MAX_NUM_TILES, 1, EMBED_DIM),
                         lambda b, c, z0, z1, ar, g: (z0, z1, z0, z1)),
        ],
        out_specs=pl.BlockSpec((1, CH, N_TILES, EMBED_DIM),
                               lambda b, c, z0, z1, ar, g: (b, c, z0, z1)),
    )

    out = pl.pallas_call(
        _body,
        grid_spec=grid_spec,
        out_shape=jax.ShapeDtypeStruct(xt.shape, xt.dtype),
        compiler_params=pltpu.CompilerParams(
            vmem_limit_bytes=60 * 1024 * 1024,
        ),
    )(ar, gate.astype(jnp.float32), xt, embedding)
    return jnp.transpose(out, (0, 2, 1, 3))
